# baseline (speedup 1.0000x reference)
import numpy as np
import jax
import jax.numpy as jnp
from jax import lax
from jax.experimental import pallas as pl
from jax.experimental.pallas import tpu as pltpu

NUM_SIGMAS = 1000
_PAD = 1024
_BB = 16


def _compute_sigmas_np():
    betas = np.linspace(0.00085 ** 0.5, 0.012 ** 0.5, 1000, dtype=np.float64) ** 2
    alphas_cumprod = np.cumprod(1.0 - betas, axis=0)
    sigmas = ((1.0 - alphas_cumprod) / alphas_cumprod) ** 0.5
    return sigmas.astype(np.float32)


_SIGMAS_ROW = np.full((1, _PAD), 1e30, dtype=np.float32)
_SIGMAS_ROW[0, :NUM_SIGMAS] = _compute_sigmas_np()

_DELTA_ROW = np.zeros((1, 128), dtype=np.float32)
_DELTA_ROW[0, [0, 5, 10, 15]] = 1.0


def _body(sigma_ref, sig_ref, wrow_ref, delta_ref, cond_ref, p_ref,
          x_ref, out_ref, coefv_ref, coefs_ref, sem):
    g = pl.program_id(0)

    @pl.when(g == 0)
    def _prologue():
        s = sigma_ref[:, :]
        sig = sig_ref[:, :]
        d = jnp.abs(s - sig)
        m = jnp.min(d, axis=1, keepdims=True)
        col = lax.broadcasted_iota(jnp.int32, d.shape, 1)
        idx = jnp.min(jnp.where(d == m, col, jnp.int32(1 << 30)), axis=1,
                      keepdims=True)
        sq = jnp.sum(jnp.where(col == idx, sig, 0.0), axis=1, keepdims=True)

        c_in = lax.rsqrt(sq * sq + 1.0)
        alpha = -sq * c_in
        beta = 1.0 + alpha * jnp.sin(idx.astype(jnp.float32) / NUM_SIGMAS)

        coefv_ref[:, :] = alpha * wrow_ref[:, :] + beta * delta_ref[:, :]
        dot = jnp.dot(cond_ref[:, :], p_ref[:, :],
                      preferred_element_type=jnp.float32)
        coefv_ref[:, 16:20] = jnp.broadcast_to(-sq, (32, 4)) * dot

        copy = pltpu.make_async_copy(coefv_ref, coefs_ref, sem)
        copy.start()
        copy.wait()

    for i in range(_BB):
        gi = g * _BB + i
        x = x_ref[i]
        for d in range(4):
            acc = x[0] * coefs_ref[gi, d]
            for c in range(1, 4):
                acc = acc + x[c] * coefs_ref[gi, 4 * c + d]
            out_ref[i, d] = acc + coefs_ref[gi, 16 + d]


def kernel(input, sigma, cond, W, P):
    B, C, H, Wd = input.shape
    D = cond.shape[1]

    sig_row = jnp.asarray(_SIGMAS_ROW)
    delta_row = jnp.asarray(_DELTA_ROW)
    wrow = jnp.pad(W.reshape(1, 16), ((0, 0), (0, 112)))

    return pl.pallas_call(
        _body,
        grid=(B // _BB,),
        in_specs=[
            pl.BlockSpec((B, 1), lambda b: (0, 0)),
            pl.BlockSpec((1, _PAD), lambda b: (0, 0)),
            pl.BlockSpec((1, 128), lambda b: (0, 0)),
            pl.BlockSpec((1, 128), lambda b: (0, 0)),
            pl.BlockSpec((B, D), lambda b: (0, 0)),
            pl.BlockSpec((D, C), lambda b: (0, 0)),
            pl.BlockSpec((_BB, C, H, Wd), lambda b: (b, 0, 0, 0)),
        ],
        out_specs=pl.BlockSpec((_BB, C, H, Wd), lambda b: (b, 0, 0, 0)),
        out_shape=jax.ShapeDtypeStruct((B, C, H, Wd), jnp.float32),
        scratch_shapes=[
            pltpu.VMEM((B, 128), jnp.float32),
            pltpu.SMEM((B, 128), jnp.float32),
            pltpu.SemaphoreType.DMA,
        ],
    )(sigma.reshape(B, 1), sig_row, wrow, delta_row, cond, P, input)


# speedup vs baseline: 1.1372x; 1.1372x over previous
import numpy as np
import jax
import jax.numpy as jnp
from jax import lax
from jax.experimental import pallas as pl
from jax.experimental.pallas import tpu as pltpu

NUM_SIGMAS = 1000
_PAD = 1024
_BB = 16


def _compute_sigmas_np():
    betas = np.linspace(0.00085 ** 0.5, 0.012 ** 0.5, 1000, dtype=np.float64) ** 2
    alphas_cumprod = np.cumprod(1.0 - betas, axis=0)
    sigmas = ((1.0 - alphas_cumprod) / alphas_cumprod) ** 0.5
    return sigmas.astype(np.float32)


_SIGMAS_ROW = np.full((1, _PAD), 1e30, dtype=np.float32)
_SIGMAS_ROW[0, :NUM_SIGMAS] = _compute_sigmas_np()


def _body(sigma_ref, sig_ref, w_ref, cond_ref, p_ref,
          x_ref, out_ref, coef_ref):
    g = pl.program_id(0)

    @pl.when(g == 0)
    def _prologue():
        s = sigma_ref[:, :]
        sig = sig_ref[:, :]
        d = jnp.abs(s - sig)
        m = jnp.min(d, axis=1, keepdims=True)
        col = lax.broadcasted_iota(jnp.int32, d.shape, 1)
        idx = jnp.min(jnp.where(d == m, col, jnp.int32(1 << 30)), axis=1,
                      keepdims=True)
        sq = jnp.sum(jnp.where(col == idx, sig, 0.0), axis=1, keepdims=True)

        c_in = lax.rsqrt(sq * sq + 1.0)
        alpha = -sq * c_in
        beta = 1.0 + alpha * jnp.sin(idx.astype(jnp.float32) / NUM_SIGMAS)

        alpha_bc = jnp.broadcast_to(alpha, (32, 128))
        beta_bc = jnp.broadcast_to(beta, (32, 128))
        sq_bc = jnp.broadcast_to(sq, (32, 128))
        for c in range(4):
            for dd in range(4):
                plane = alpha_bc * w_ref[c, dd]
                if c == dd:
                    plane = plane + beta_bc
                coef_ref[4 * c + dd] = plane
        coef_ref[16] = sq_bc

    dot = jnp.dot(cond_ref[:, :], p_ref[:, :],
                  preferred_element_type=jnp.float32)
    nsq = -coef_ref[16, pl.ds(g * _BB, _BB), 0:4]
    bias = nsq * dot
    bias_bc = [jnp.broadcast_to(bias[:, d:d + 1], (_BB, 128))
               for d in range(4)]

    for i in range(_BB):
        gi = g * _BB + i
        x = x_ref[i]
        for d in range(4):
            acc = x[0] * coef_ref[d, pl.ds(gi, 1), :]
            for c in range(1, 4):
                acc = acc + x[c] * coef_ref[4 * c + d, pl.ds(gi, 1), :]
            out_ref[i, d] = acc + bias_bc[d][i:i + 1, :]


def kernel(input, sigma, cond, W, P):
    B, C, H, Wd = input.shape
    D = cond.shape[1]

    sig_row = jnp.asarray(_SIGMAS_ROW)

    return pl.pallas_call(
        _body,
        grid=(B // _BB,),
        in_specs=[
            pl.BlockSpec((B, 1), lambda b: (0, 0)),
            pl.BlockSpec((1, _PAD), lambda b: (0, 0)),
            pl.BlockSpec(memory_space=pltpu.SMEM),
            pl.BlockSpec((_BB, D), lambda b: (b, 0)),
            pl.BlockSpec((D, C), lambda b: (0, 0)),
            pl.BlockSpec((_BB, C, H, Wd), lambda b: (b, 0, 0, 0)),
        ],
        out_specs=pl.BlockSpec((_BB, C, H, Wd), lambda b: (b, 0, 0, 0)),
        out_shape=jax.ShapeDtypeStruct((B, C, H, Wd), jnp.float32),
        scratch_shapes=[pltpu.VMEM((17, B, 128), jnp.float32)],
    )(sigma.reshape(B, 1), sig_row, W, cond, P, input)


# speedup vs baseline: 1.1522x; 1.0132x over previous
import numpy as np
import jax
import jax.numpy as jnp
from jax import lax
from jax.experimental import pallas as pl
from jax.experimental.pallas import tpu as pltpu

NUM_SIGMAS = 1000
_PAD = 1024
_BB = 16


def _compute_sigmas_np():
    betas = np.linspace(0.00085 ** 0.5, 0.012 ** 0.5, 1000, dtype=np.float64) ** 2
    alphas_cumprod = np.cumprod(1.0 - betas, axis=0)
    sigmas = ((1.0 - alphas_cumprod) / alphas_cumprod) ** 0.5
    return sigmas.astype(np.float32)


_SIGMAS_ROW = np.full((1, _PAD), 1e30, dtype=np.float32)
_SIGMAS_ROW[0, :NUM_SIGMAS] = _compute_sigmas_np()

_MID_ROW = np.full((1, _PAD), 1e30, dtype=np.float32)
_s64 = _compute_sigmas_np().astype(np.float64)
_MID_ROW[0, :NUM_SIGMAS - 1] = ((_s64[:-1] + _s64[1:]) * 0.5).astype(np.float32)


def _body(sigma_ref, sig_ref, mid_ref, w_ref, cond_ref, p_ref,
          x_ref, out_ref, coef_ref):
    g = pl.program_id(0)

    @pl.when(g == 0)
    def _prologue():
        s = sigma_ref[:, :]
        sig = sig_ref[:, :]
        mid = mid_ref[:, :]
        cntf = jnp.sum((mid < s).astype(jnp.float32), axis=1,
                       keepdims=True)
        idx = cntf.astype(jnp.int32)
        col = lax.broadcasted_iota(jnp.int32, (32, _PAD), 1)
        sq = jnp.sum(jnp.where(col == idx, sig, 0.0), axis=1, keepdims=True)

        c_in = lax.rsqrt(sq * sq + 1.0)
        alpha = -sq * c_in
        beta = 1.0 + alpha * jnp.sin(cntf / NUM_SIGMAS)

        alpha_bc = jnp.broadcast_to(alpha, (32, 128))
        beta_bc = jnp.broadcast_to(beta, (32, 128))
        sq_bc = jnp.broadcast_to(sq, (32, 128))
        for c in range(4):
            for dd in range(4):
                plane = alpha_bc * w_ref[c, dd]
                if c == dd:
                    plane = plane + beta_bc
                coef_ref[4 * c + dd] = plane
        coef_ref[16] = sq_bc

    dot = jnp.dot(cond_ref[:, :], p_ref[:, :],
                  preferred_element_type=jnp.float32)
    nsq = -coef_ref[16, pl.ds(g * _BB, _BB), 0:4]
    bias = nsq * dot
    bias_bc = [jnp.broadcast_to(bias[:, d:d + 1], (_BB, 128))
               for d in range(4)]

    for i in range(_BB):
        gi = g * _BB + i
        x = x_ref[i]
        for d in range(4):
            acc = x[0] * coef_ref[d, pl.ds(gi, 1), :]
            for c in range(1, 4):
                acc = acc + x[c] * coef_ref[4 * c + d, pl.ds(gi, 1), :]
            out_ref[i, d] = acc + bias_bc[d][i:i + 1, :]


def kernel(input, sigma, cond, W, P):
    B, C, H, Wd = input.shape
    D = cond.shape[1]

    sig_row = jnp.asarray(_SIGMAS_ROW)
    mid_row = jnp.asarray(_MID_ROW)

    return pl.pallas_call(
        _body,
        grid=(B // _BB,),
        in_specs=[
            pl.BlockSpec((B, 1), lambda b: (0, 0)),
            pl.BlockSpec((1, _PAD), lambda b: (0, 0)),
            pl.BlockSpec((1, _PAD), lambda b: (0, 0)),
            pl.BlockSpec(memory_space=pltpu.SMEM),
            pl.BlockSpec((_BB, D), lambda b: (b, 0)),
            pl.BlockSpec((D, C), lambda b: (0, 0)),
            pl.BlockSpec((_BB, C, H, Wd), lambda b: (b, 0, 0, 0)),
        ],
        out_specs=pl.BlockSpec((_BB, C, H, Wd), lambda b: (b, 0, 0, 0)),
        out_shape=jax.ShapeDtypeStruct((B, C, H, Wd), jnp.float32),
        scratch_shapes=[pltpu.VMEM((17, B, 128), jnp.float32)],
    )(sigma.reshape(B, 1), sig_row, mid_row, W, cond, P, input)
